# Initial kernel scaffold; baseline (speedup 1.0000x reference)
import jax
import jax.numpy as jnp
from jax.experimental import pallas as pl


def kernel(h, us, ue, u, W_non_noise, b_non_noise, W_noise, b_noise, W_E, b_E, W_r, b_r):
    raise NotImplementedError("write your pallas kernel here")


# speedup vs baseline: 2.2920x; 2.2920x over previous
import jax
import jax.numpy as jnp
import numpy as np
from jax.experimental import pallas as pl
from jax.experimental.pallas import tpu as pltpu

_S = 2048
_D = 768
_E = 8
_KU = 2 * _D
_T = 256
_DC = 128
_NC = _D // _DC
_NT = _S // _T
_W8 = _E * _DC
_BN = 3 * _W8

_NOISE_P = np.asarray(
    jax.random.normal(jax.random.key(12345), (1, _S, _D, _E), dtype=jnp.float32)
).reshape(_S, _NC, _DC, _E).transpose(0, 1, 3, 2).reshape(_S, _D * _E)


def _bias_kernel(hcat_ref, wr_ref, br_ref, wrall_ref, ball_ref, out_ref, r8):
    @pl.when(pl.program_id(0) == 0)
    def _():
        r8[...] = (
            jnp.dot(hcat_ref[...], wr_ref[...], preferred_element_type=jnp.float32)
            + br_ref[...]
        )

    out_ref[...] = (
        jnp.dot(r8[...], wrall_ref[...], preferred_element_type=jnp.float32)
        + ball_ref[...]
    )


def _main_kernel(x_ref, w_ref, beff_ref, nz_ref, out_ref):
    y = (
        jnp.dot(x_ref[...], w_ref[...], preferred_element_type=jnp.float32)
        + beff_ref[0][None, :]
    )
    y_nn = y[:, : _W8]
    y_no = y[:, _W8 : 2 * _W8]
    y_e = y[:, 2 * _W8 :]
    hs = y_nn + y_no * nz_ref[...]

    m1 = hs[:, :_DC]
    e1 = y_e[:, :_DC]
    m2 = jnp.full_like(m1, -jnp.inf)
    e2 = jnp.zeros_like(e1)
    for i in range(1, _E):
        v = hs[:, i * _DC : (i + 1) * _DC]
        ee = y_e[:, i * _DC : (i + 1) * _DC]
        gt = v > m1
        cand = jnp.where(gt, m1, v)
        cand_e = jnp.where(gt, e1, ee)
        m1 = jnp.where(gt, v, m1)
        e1 = jnp.where(gt, ee, e1)
        gt2 = cand > m2
        m2 = jnp.where(gt2, cand, m2)
        e2 = jnp.where(gt2, cand_e, e2)

    s = jnp.exp(m2 - m1)
    out_ref[...] = (e1 + e2 * s) / (1.0 + s) * (1.0 / _E)


def kernel(h, us, ue, u, W_non_noise, b_non_noise, W_noise, b_noise, W_E, b_E, W_r, b_r):
    f32 = jnp.float32

    w_all = jnp.stack([W_non_noise, W_noise, W_E], axis=0)
    w_all = w_all.reshape(3, 3 * _D, _NC, _DC, _E)
    w_all = w_all.transpose(1, 2, 0, 4, 3).reshape(3 * _D, 3 * _D * _E)
    w_u = w_all[:_KU]
    w_r_part = w_all[_KU:]

    b_all = jnp.stack([b_non_noise, b_noise, b_E], axis=0)
    b_all = b_all.reshape(3, _NC, _DC, _E).transpose(1, 0, 3, 2).reshape(3 * _D * _E)
    b_all8 = jnp.broadcast_to(b_all[None, :], (8, 3 * _D * _E))

    hcat8 = jnp.broadcast_to(
        jnp.concatenate([h, us, ue], axis=-1).reshape(1, 5 * _D), (8, 5 * _D)
    )
    br8 = jnp.broadcast_to(b_r[None, :], (8, _D))
    x2d = u.reshape(_S, _KU)

    beff = pl.pallas_call(
        _bias_kernel,
        grid=(_NC,),
        in_specs=[
            pl.BlockSpec((8, 5 * _D), lambda c: (0, 0)),
            pl.BlockSpec((5 * _D, _D), lambda c: (0, 0)),
            pl.BlockSpec((8, _D), lambda c: (0, 0)),
            pl.BlockSpec((_D, _BN), lambda c: (0, c)),
            pl.BlockSpec((8, _BN), lambda c: (0, c)),
        ],
        out_specs=pl.BlockSpec((8, _BN), lambda c: (0, c)),
        out_shape=jax.ShapeDtypeStruct((8, 3 * _D * _E), f32),
        scratch_shapes=[pltpu.VMEM((8, _D), f32)],
    )(hcat8, W_r, br8, w_r_part, b_all8)

    out2d = pl.pallas_call(
        _main_kernel,
        grid=(_NC, _NT),
        in_specs=[
            pl.BlockSpec((_T, _KU), lambda c, t: (t, 0)),
            pl.BlockSpec((_KU, _BN), lambda c, t: (0, c)),
            pl.BlockSpec((8, _BN), lambda c, t: (0, c)),
            pl.BlockSpec((_T, _W8), lambda c, t: (t, c)),
        ],
        out_specs=pl.BlockSpec((_T, _DC), lambda c, t: (t, c)),
        out_shape=jax.ShapeDtypeStruct((_S, _D), f32),
    )(x2d, w_u, beff, jnp.asarray(_NOISE_P))

    return out2d.reshape(1, _S, _D)


# speedup vs baseline: 4.7744x; 2.0831x over previous
import jax
import jax.numpy as jnp
import numpy as np
from jax import lax
from jax.experimental import pallas as pl
from jax.experimental.pallas import tpu as pltpu

_S = 2048
_D = 768
_E = 8
_KH = _D
_T = 256
_DC = 128
_NC = _D // _DC
_NT = _S // _T
_BN = _E * _DC

_NOISE = np.asarray(
    jax.random.normal(jax.random.key(12345), (1, _S, _D, _E), dtype=jnp.float32)
).reshape(_S, _D * _E)

_SSUM = np.zeros((_BN, _DC), dtype=np.float32)
_SSUM[np.arange(_BN), np.arange(_BN) // _E] = 1.0 / _E


def _rotg(v, s):
    pos = lax.broadcasted_iota(jnp.int32, v.shape, 1) % _E
    return jnp.where(pos < _E - s,
                     pltpu.roll(v, v.shape[1] - s, axis=1),
                     pltpu.roll(v, _E - s, axis=1))


def _gmax(v):
    for s in (1, 2, 4):
        v = jnp.maximum(v, _rotg(v, s))
    return v


def _gmin(v):
    for s in (1, 2, 4):
        v = jnp.minimum(v, _rotg(v, s))
    return v


def _bias_kernel(hcat_ref, wr_ref, br_ref, wnn_ref, wno_ref, we_ref,
                 bnn_ref, bno_ref, be_ref, onn_ref, ono_ref, oe_ref, r8):
    @pl.when(pl.program_id(0) == 0)
    def _():
        r8[...] = (
            jnp.dot(hcat_ref[...], wr_ref[...], preferred_element_type=jnp.float32)
            + br_ref[...]
        )

    r = r8[...]
    onn_ref[...] = jnp.dot(r, wnn_ref[...], preferred_element_type=jnp.float32) + bnn_ref[...]
    ono_ref[...] = jnp.dot(r, wno_ref[...], preferred_element_type=jnp.float32) + bno_ref[...]
    oe_ref[...] = jnp.dot(r, we_ref[...], preferred_element_type=jnp.float32) + be_ref[...]


def _main_kernel(x_ref, wnnl_ref, wnnh_ref, wnol_ref, wnoh_ref, wel_ref, weh_ref,
                 bnn_ref, bno_ref, be_ref, nz_ref, ssum_ref, out_ref):
    f32 = jnp.float32
    xl = x_ref[:, :_KH]
    xh = x_ref[:, _KH:]
    y_nn = (jnp.dot(xl, wnnl_ref[...], preferred_element_type=f32)
            + jnp.dot(xh, wnnh_ref[...], preferred_element_type=f32)
            + bnn_ref[0][None, :])
    y_no = (jnp.dot(xl, wnol_ref[...], preferred_element_type=f32)
            + jnp.dot(xh, wnoh_ref[...], preferred_element_type=f32)
            + bno_ref[0][None, :])
    y_e = (jnp.dot(xl, wel_ref[...], preferred_element_type=f32)
           + jnp.dot(xh, weh_ref[...], preferred_element_type=f32)
           + be_ref[0][None, :])
    hs = y_nn + y_no * nz_ref[...]

    pos = lax.broadcasted_iota(jnp.int32, hs.shape, 1) % _E
    m1 = _gmax(hs)
    fm = _gmin(jnp.where(hs == m1, pos, _E))
    sel1 = pos == fm
    v2 = jnp.where(sel1, -jnp.inf, hs)
    m2 = _gmax(v2)
    fm2 = _gmin(jnp.where(v2 == m2, pos, _E))
    s = jnp.exp(m2 - m1)
    inv_z = 1.0 / (1.0 + s)
    g = jnp.where(sel1, inv_z, jnp.where(pos == fm2, s * inv_z, 0.0))
    out_ref[...] = jnp.dot(g * y_e, ssum_ref[...], preferred_element_type=f32)


def kernel(h, us, ue, u, W_non_noise, b_non_noise, W_noise, b_noise, W_E, b_E, W_r, b_r):
    f32 = jnp.float32

    hcat8 = jnp.broadcast_to(
        jnp.concatenate([h, us, ue], axis=-1).reshape(1, 5 * _D), (8, 5 * _D)
    )
    br8 = jnp.broadcast_to(b_r[None, :], (8, _D))
    bnn8 = jnp.broadcast_to(b_non_noise[None, :], (8, _D * _E))
    bno8 = jnp.broadcast_to(b_noise[None, :], (8, _D * _E))
    be8 = jnp.broadcast_to(b_E[None, :], (8, _D * _E))
    x2d = u.reshape(_S, 2 * _D)

    row2 = pl.BlockSpec((_KH, _BN), lambda c: (2, c))
    bspec = pl.BlockSpec((8, _BN), lambda c: (0, c))
    beff_nn, beff_no, beff_e = pl.pallas_call(
        _bias_kernel,
        grid=(_NC,),
        in_specs=[
            pl.BlockSpec((8, 5 * _D), lambda c: (0, 0)),
            pl.BlockSpec((5 * _D, _D), lambda c: (0, 0)),
            pl.BlockSpec((8, _D), lambda c: (0, 0)),
            row2, row2, row2,
            bspec, bspec, bspec,
        ],
        out_specs=[bspec, bspec, bspec],
        out_shape=[jax.ShapeDtypeStruct((8, _D * _E), f32)] * 3,
        scratch_shapes=[pltpu.VMEM((8, _D), f32)],
    )(hcat8, W_r, br8, W_non_noise, W_noise, W_E, bnn8, bno8, be8)

    row0 = pl.BlockSpec((_KH, _BN), lambda c, t: (0, c))
    row1 = pl.BlockSpec((_KH, _BN), lambda c, t: (1, c))
    bspec2 = pl.BlockSpec((8, _BN), lambda c, t: (0, c))
    out2d = pl.pallas_call(
        _main_kernel,
        grid=(_NC, _NT),
        in_specs=[
            pl.BlockSpec((_T, 2 * _D), lambda c, t: (t, 0)),
            row0, row1, row0, row1, row0, row1,
            bspec2, bspec2, bspec2,
            pl.BlockSpec((_T, _BN), lambda c, t: (t, c)),
            pl.BlockSpec((_BN, _DC), lambda c, t: (0, 0)),
        ],
        out_specs=pl.BlockSpec((_T, _DC), lambda c, t: (t, c)),
        out_shape=jax.ShapeDtypeStruct((_S, _D), f32),
    )(x2d, W_non_noise, W_non_noise, W_noise, W_noise, W_E, W_E,
      beff_nn, beff_no, beff_e, jnp.asarray(_NOISE), jnp.asarray(_SSUM))

    return out2d.reshape(1, _S, _D)
